# Initial kernel scaffold; baseline (speedup 1.0000x reference)
import jax
import jax.numpy as jnp
from jax.experimental import pallas as pl


def kernel(x, edge_index, edge_attr, adj, Wl1, bl1, Wr1, br1, Wl2, bl2, Wr2, br2, Wg1, bg1, sag_Wrel, sag_brel, sag_Wroot, cheb_W0, cheb_W1, cheb_W2, cheb_b):
    raise NotImplementedError("write your pallas kernel here")


# speedup vs baseline: 16.6778x; 16.6778x over previous
import functools
import numpy as np
import jax
import jax.numpy as jnp
from jax.experimental import pallas as pl

N_NODES = 111
N_EDGES = 6000
NP = 128
EP = 6144
_LEFT = np.array([6,5,55,1,98,71,73,77,63,96,79,15,104,4,25,23,41,43,45,17,61,65,59,57,86,21,35,37,39,94,110,3,69,81,84,100,102,106,47,27,75,2,67,19,49,31,33,108,51,53,88,90,92,29,0], dtype=np.int32)
_RIGHT = np.array([13,12,54,8,97,70,72,76,62,95,78,14,103,11,24,22,40,42,44,16,60,64,58,56,85,20,34,36,38,93,109,10,68,80,83,99,101,105,46,26,74,9,66,18,48,30,32,107,50,52,87,89,91,28,7], dtype=np.int32)

_HI = jax.lax.Precision.HIGHEST


def _fused_body(xp, srcp, dstp, ewp, sidemat, lcol, rcol, wl1, wr1, wl2, wr2,
                wg1, w0, w1, w2, biasm, out_ref):
    f32 = jnp.float32
    x = xp[...]
    src = srcp[...]
    dst = dstp[...]
    ew = ewp[...]
    lanes_e = jax.lax.broadcasted_iota(jnp.int32, (EP, NP), 1)
    eidx = jax.lax.broadcasted_iota(jnp.int32, (EP, 1), 0)
    Es = (src == lanes_e).astype(f32)
    Ed = (dst == lanes_e).astype(f32)
    evalid = (eidx < N_EDGES).astype(f32)

    smat = sidemat[...]
    memb_s = jax.lax.dot(Es, smat, precision=_HI)
    memb_d = jax.lax.dot(Ed, smat, precision=_HI)
    lmask = memb_s[:, 0:1] * memb_d[:, 0:1]
    rmask = memb_s[:, 1:2] * memb_d[:, 1:2]

    def scat(v):
        return jax.lax.dot_general(Ed * v, Es, (((0,), (0,)), ((), ())),
                                   precision=_HI)

    A_l = scat(ew * lmask)
    A_r = scat(ew * rmask)
    A_g = scat(ew * evalid)
    A_c = scat(evalid)

    def gcn(A, xw):
        deg = A.sum(axis=1, keepdims=True) + 1.0
        dis = jnp.where(deg > 0, 1.0 / jnp.sqrt(jnp.abs(deg) + 1e-30), 0.0)
        return dis * (jax.lax.dot(A, dis * xw, precision=_HI) + dis * xw)

    def lrelu(t):
        return jnp.where(t >= 0, t, 0.01 * t)

    bias = biasm[...]
    lv = lcol[...]
    rv = rcol[...]

    h_l = lrelu(gcn(A_l, jax.lax.dot(x, wl1[...])) + bias[0:1, :])
    h_r = lrelu(gcn(A_r, jax.lax.dot(x, wr1[...])) + bias[1:2, :])
    h1 = lv * h_l + rv * h_r
    h_l2 = lrelu(gcn(A_l, jax.lax.dot(h1, wl2[...])) + bias[2:3, :])
    h_r2 = lrelu(gcn(A_r, jax.lax.dot(h1, wr2[...])) + bias[3:4, :])
    h2 = lv * h_l2 + rv * h_r2
    h2 = lrelu(gcn(A_g, jax.lax.dot(h2, wg1[...])) + bias[4:5, :])

    agg = jax.lax.dot(A_c, h2, precision=_HI)
    brel = bias[8:9, 0:1]
    raw = (jax.lax.dot(agg, smat)[:, 2:3] + brel
           + jax.lax.dot(h2, smat)[:, 3:4])
    t = jnp.exp(-2.0 * jnp.abs(raw))
    score = jnp.sign(raw) * (1.0 - t) / (1.0 + t)
    rowid = jax.lax.broadcasted_iota(jnp.int32, (NP, 1), 0)
    score = jnp.where(rowid < N_NODES, score, -3e38)

    sc_r = jnp.transpose(score)
    gt = (sc_r > score).astype(f32)
    eqlt = ((sc_r == score) & (jnp.transpose(rowid) < rowid)).astype(f32)
    rank = (gt + eqlt).sum(axis=1, keepdims=True)
    rowf = rowid.astype(f32)
    Pfull = (jnp.transpose(rank) == rowf).astype(f32)
    topv = (Pfull * jnp.transpose(score)).sum(axis=1, keepdims=True)
    selrank = (rowid < 100).astype(f32)
    pooled = jax.lax.dot(Pfull, h2, precision=_HI) * topv * selrank

    sel = (rank < 100).astype(f32)
    degf = jnp.transpose(sel) * (A_c * sel).sum(axis=0, keepdims=True)
    degv = (Pfull * degf).sum(axis=1, keepdims=True)
    disv = jnp.where(degv > 0, 1.0 / jnp.sqrt(jnp.abs(degv) + 1e-30), 0.0)
    C = A_c * sel * jnp.transpose(sel)
    Cp = jax.lax.dot_general(jax.lax.dot(Pfull, C, precision=_HI), Pfull,
                             (((1,), (1,)), ((), ())), precision=_HI)
    L = -(disv * jnp.transpose(disv)) * Cp

    Tx1 = jax.lax.dot(L, h2, precision=_HI)
    Tx2 = 2.0 * jax.lax.dot(L, Tx1, precision=_HI) - h2
    cheb_out = (jax.lax.dot(h2, w0[...])
                + jax.lax.dot(Tx1, w1[...])
                + jax.lax.dot(Tx2, w2[...]) + bias[5:6, :])

    colm = (jax.lax.broadcasted_iota(jnp.int32, (1, NP), 1) < 56).astype(f32)

    def msoftmax(z):
        zm = jnp.where(colm > 0, z, -3e38)
        e = jnp.exp(zm - zm.max(axis=1, keepdims=True)) * colm
        return e / e.sum(axis=1, keepdims=True)

    ass = msoftmax(cheb_out)
    s = msoftmax(ass) * (rowid < N_NODES).astype(f32)
    H_coarse = jax.lax.dot_general(s, h2, (((0,), (0,)), ((), ())))

    sel2 = sel * (rowid < 110).astype(f32)
    tri = (jnp.transpose(rowid) < rowid).astype(f32)
    srank = (tri * jnp.transpose(sel2)).sum(axis=1, keepdims=True)
    Q = jnp.transpose(sel2) * (jnp.transpose(srank) == rowf).astype(f32)
    rows = jax.lax.dot(Q, ass, precision=_HI)
    H1 = jax.lax.dot(rows, H_coarse)
    out_ref[...] = pooled + H1


@jax.jit
def kernel(x, edge_index, edge_attr, adj, Wl1, bl1, Wr1, br1, Wl2, bl2, Wr2,
           br2, Wg1, bg1, sag_Wrel, sag_brel, sag_Wroot, cheb_W0, cheb_W1,
           cheb_W2, cheb_b):
    f32 = jnp.float32
    pe = EP - N_EDGES
    srcp = jnp.concatenate([edge_index[0], jnp.full((pe,), 120, jnp.int32)])
    dstp = jnp.concatenate([edge_index[1], jnp.full((pe,), 120, jnp.int32)])
    ewp = jnp.concatenate([edge_attr, jnp.zeros((pe,), f32)])

    def pad2(a):
        return jnp.zeros((NP, NP), f32).at[:a.shape[0], :a.shape[1]].set(a)

    sideL = np.zeros((NP,), np.float32); sideL[_LEFT] = 1.0
    sideR = np.zeros((NP,), np.float32); sideR[_RIGHT] = 1.0
    sidemat = np.zeros((NP, NP), np.float32)
    sidemat[:, 0] = sideL; sidemat[:, 1] = sideR
    sidemat = jnp.asarray(sidemat)
    sidemat = sidemat.at[:20, 2].set(sag_Wrel[:, 0])
    sidemat = sidemat.at[:20, 3].set(sag_Wroot[:, 0])

    biasm = jnp.zeros((16, NP), f32)
    biasm = biasm.at[0, :64].set(bl1).at[1, :64].set(br1)
    biasm = biasm.at[2, :20].set(bl2).at[3, :20].set(br2)
    biasm = biasm.at[4, :20].set(bg1).at[5, :56].set(cheb_b)
    biasm = biasm.at[6, :20].set(sag_Wrel[:, 0]).at[7, :20].set(sag_Wroot[:, 0])
    biasm = biasm.at[8, 0].set(sag_brel[0])

    out = pl.pallas_call(
        _fused_body,
        out_shape=jax.ShapeDtypeStruct((NP, NP), f32),
    )(
        pad2(x), srcp[:, None], dstp[:, None], ewp[:, None],
        jnp.asarray(sidemat), jnp.asarray(sideL)[:, None],
        jnp.asarray(sideR)[:, None], pad2(Wl1), pad2(Wr1), pad2(Wl2),
        pad2(Wr2), pad2(Wg1), pad2(cheb_W0), pad2(cheb_W1), pad2(cheb_W2),
        biasm,
    )
    return out[:100, :20].reshape(1, -1)


# speedup vs baseline: 20.2571x; 1.2146x over previous
import functools
import numpy as np
import jax
import jax.numpy as jnp
from jax.experimental import pallas as pl

N_NODES = 111
N_EDGES = 6000
NP = 128
EP = 6144
_LEFT = np.array([6,5,55,1,98,71,73,77,63,96,79,15,104,4,25,23,41,43,45,17,61,65,59,57,86,21,35,37,39,94,110,3,69,81,84,100,102,106,47,27,75,2,67,19,49,31,33,108,51,53,88,90,92,29,0], dtype=np.int32)
_RIGHT = np.array([13,12,54,8,97,70,72,76,62,95,78,14,103,11,24,22,40,42,44,16,60,64,58,56,85,20,34,36,38,93,109,10,68,80,83,99,101,105,46,26,74,9,66,18,48,30,32,107,50,52,87,89,91,28,7], dtype=np.int32)

_HI = jax.lax.Precision.HIGHEST


def _fused_body(xp, srcp, dstp, ewp, sidemat, lcol, rcol, wl1, wr1, wl2, wr2,
                wg1, w0, w1, w2, biasm, out_ref):
    f32 = jnp.float32
    bf16 = jnp.bfloat16
    x = xp[...]
    src = srcp[...]
    dst = dstp[...]
    ew = ewp[...]
    smat = sidemat[...]
    lanes_e = jax.lax.broadcasted_iota(jnp.int32, (EP, NP), 1)
    eidx = jax.lax.broadcasted_iota(jnp.int32, (EP, 1), 0)
    Es = (src == lanes_e).astype(f32).astype(bf16)
    edm = dst == lanes_e
    Ed = edm.astype(f32).astype(bf16)
    Edc = (edm & (eidx < N_EDGES)).astype(f32).astype(bf16)
    vh = ew.astype(bf16)
    r1 = ew - vh.astype(f32)
    vm = r1.astype(bf16)
    vl = (r1 - vm.astype(f32)).astype(bf16)
    EdV = jnp.concatenate([Ed * vh, Ed * vm, Ed * vl, Edc], axis=1)
    out4 = jax.lax.dot_general(EdV, Es, (((0,), (0,)), ((), ())),
                               preferred_element_type=f32)
    A_g = out4[0:NP] + out4[NP:2 * NP] + out4[2 * NP:3 * NP]
    A_c = out4[3 * NP:4 * NP]
    lvc = lcol[...]
    rvc = rcol[...]
    A_l = A_g * lvc * jnp.transpose(lvc)
    A_r = A_g * rvc * jnp.transpose(rvc)

    def gcn(A, xw):
        deg = A.sum(axis=1, keepdims=True) + 1.0
        dis = jnp.where(deg > 0, 1.0 / jnp.sqrt(jnp.abs(deg) + 1e-30), 0.0)
        return dis * (jax.lax.dot(A, dis * xw, precision=_HI) + dis * xw)

    def lrelu(t):
        return jnp.where(t >= 0, t, 0.01 * t)

    bias = biasm[...]
    lv = lvc
    rv = rvc

    h_l = lrelu(gcn(A_l, jax.lax.dot(x, wl1[...])) + bias[0:1, :])
    h_r = lrelu(gcn(A_r, jax.lax.dot(x, wr1[...])) + bias[1:2, :])
    h1 = lv * h_l + rv * h_r
    h_l2 = lrelu(gcn(A_l, jax.lax.dot(h1, wl2[...])) + bias[2:3, :])
    h_r2 = lrelu(gcn(A_r, jax.lax.dot(h1, wr2[...])) + bias[3:4, :])
    h2 = lv * h_l2 + rv * h_r2
    h2 = lrelu(gcn(A_g, jax.lax.dot(h2, wg1[...])) + bias[4:5, :])

    agg = jax.lax.dot(A_c, h2, precision=_HI)
    brel = bias[8:9, 0:1]
    raw = (jax.lax.dot(agg, smat)[:, 2:3] + brel
           + jax.lax.dot(h2, smat)[:, 3:4])
    t = jnp.exp(-2.0 * jnp.abs(raw))
    score = jnp.sign(raw) * (1.0 - t) / (1.0 + t)
    rowid = jax.lax.broadcasted_iota(jnp.int32, (NP, 1), 0)
    score = jnp.where(rowid < N_NODES, score, -3e38)

    sc_r = jnp.transpose(score)
    gt = (sc_r > score).astype(f32)
    eqlt = ((sc_r == score) & (jnp.transpose(rowid) < rowid)).astype(f32)
    rank = (gt + eqlt).sum(axis=1, keepdims=True)
    rowf = rowid.astype(f32)
    Pfull = (jnp.transpose(rank) == rowf).astype(f32)
    topv = (Pfull * jnp.transpose(score)).sum(axis=1, keepdims=True)
    selrank = (rowid < 100).astype(f32)
    pooled = jax.lax.dot(Pfull, h2, precision=_HI) * topv * selrank

    sel = (rank < 100).astype(f32)
    degf = jnp.transpose(sel) * (A_c * sel).sum(axis=0, keepdims=True)
    degv = (Pfull * degf).sum(axis=1, keepdims=True)
    disv = jnp.where(degv > 0, 1.0 / jnp.sqrt(jnp.abs(degv) + 1e-30), 0.0)
    C = A_c * sel * jnp.transpose(sel)
    Cp = jax.lax.dot_general(jax.lax.dot(Pfull, C, precision=_HI), Pfull,
                             (((1,), (1,)), ((), ())), precision=_HI)
    L = -(disv * jnp.transpose(disv)) * Cp

    Tx1 = jax.lax.dot(L, h2, precision=_HI)
    Tx2 = 2.0 * jax.lax.dot(L, Tx1, precision=_HI) - h2
    cheb_out = (jax.lax.dot(h2, w0[...])
                + jax.lax.dot(Tx1, w1[...])
                + jax.lax.dot(Tx2, w2[...]) + bias[5:6, :])

    colm = (jax.lax.broadcasted_iota(jnp.int32, (1, NP), 1) < 56).astype(f32)

    def msoftmax(z):
        zm = jnp.where(colm > 0, z, -3e38)
        e = jnp.exp(zm - zm.max(axis=1, keepdims=True)) * colm
        return e / e.sum(axis=1, keepdims=True)

    ass = msoftmax(cheb_out)
    s = msoftmax(ass) * (rowid < N_NODES).astype(f32)
    H_coarse = jax.lax.dot_general(s, h2, (((0,), (0,)), ((), ())))

    sel2 = sel * (rowid < 110).astype(f32)
    tri = (jnp.transpose(rowid) < rowid).astype(f32)
    srank = (tri * jnp.transpose(sel2)).sum(axis=1, keepdims=True)
    Q = jnp.transpose(sel2) * (jnp.transpose(srank) == rowf).astype(f32)
    rows = jax.lax.dot(Q, ass, precision=_HI)
    H1 = jax.lax.dot(rows, H_coarse)
    out_ref[...] = pooled + H1


@jax.jit
def kernel(x, edge_index, edge_attr, adj, Wl1, bl1, Wr1, br1, Wl2, bl2, Wr2,
           br2, Wg1, bg1, sag_Wrel, sag_brel, sag_Wroot, cheb_W0, cheb_W1,
           cheb_W2, cheb_b):
    f32 = jnp.float32
    pe = EP - N_EDGES
    srcp = jnp.concatenate([edge_index[0], jnp.full((pe,), 120, jnp.int32)])
    dstp = jnp.concatenate([edge_index[1], jnp.full((pe,), 120, jnp.int32)])
    ewp = jnp.concatenate([edge_attr, jnp.zeros((pe,), f32)])

    def pad2(a):
        return jnp.zeros((NP, NP), f32).at[:a.shape[0], :a.shape[1]].set(a)

    sideL = np.zeros((NP,), np.float32); sideL[_LEFT] = 1.0
    sideR = np.zeros((NP,), np.float32); sideR[_RIGHT] = 1.0
    sidemat = np.zeros((NP, NP), np.float32)
    sidemat[:, 0] = sideL; sidemat[:, 1] = sideR
    sidemat = jnp.asarray(sidemat)
    sidemat = sidemat.at[:20, 2].set(sag_Wrel[:, 0])
    sidemat = sidemat.at[:20, 3].set(sag_Wroot[:, 0])

    biasm = jnp.zeros((16, NP), f32)
    biasm = biasm.at[0, :64].set(bl1).at[1, :64].set(br1)
    biasm = biasm.at[2, :20].set(bl2).at[3, :20].set(br2)
    biasm = biasm.at[4, :20].set(bg1).at[5, :56].set(cheb_b)
    biasm = biasm.at[6, :20].set(sag_Wrel[:, 0]).at[7, :20].set(sag_Wroot[:, 0])
    biasm = biasm.at[8, 0].set(sag_brel[0])

    out = pl.pallas_call(
        _fused_body,
        out_shape=jax.ShapeDtypeStruct((NP, NP), f32),
    )(
        pad2(x), srcp[:, None], dstp[:, None], ewp[:, None],
        jnp.asarray(sidemat), jnp.asarray(sideL)[:, None],
        jnp.asarray(sideR)[:, None], pad2(Wl1), pad2(Wr1), pad2(Wl2),
        pad2(Wr2), pad2(Wg1), pad2(cheb_W0), pad2(cheb_W1), pad2(cheb_W2),
        biasm,
    )
    return out[:100, :20].reshape(1, -1)


# speedup vs baseline: 23.1858x; 1.1446x over previous
import functools
import numpy as np
import jax
import jax.numpy as jnp
from jax import lax
from jax.experimental import pallas as pl
from jax.experimental.pallas import tpu as pltpu
from jax.experimental.pallas import tpu_sc as plsc

N_NODES = 111
N_EDGES = 6000
NP = 128
EP = 6144
NWORK = 32
EPW = EP // NWORK
_LEFT = np.array([6,5,55,1,98,71,73,77,63,96,79,15,104,4,25,23,41,43,45,17,61,65,59,57,86,21,35,37,39,94,110,3,69,81,84,100,102,106,47,27,75,2,67,19,49,31,33,108,51,53,88,90,92,29,0], dtype=np.int32)
_RIGHT = np.array([13,12,54,8,97,70,72,76,62,95,78,14,103,11,24,22,40,42,44,16,60,64,58,56,85,20,34,36,38,93,109,10,68,80,83,99,101,105,46,26,74,9,66,18,48,30,32,107,50,52,87,89,91,28,7], dtype=np.int32)

_HI = jax.lax.Precision.HIGHEST


def _vg(v, idx):
    return lax.gather(
        v, idx[:, None],
        lax.GatherDimensionNumbers(offset_dims=(), collapsed_slice_dims=(0,),
                                   start_index_map=(0,)),
        (1,), mode=lax.GatherScatterMode.PROMISE_IN_BOUNDS)


@functools.partial(
    pl.kernel,
    mesh=plsc.VectorSubcoreMesh(core_axis_name="c", subcore_axis_name="s"),
    out_type=jax.ShapeDtypeStruct((2, 240, NP), jnp.float32),
    scratch_types=[
        pltpu.VMEM((EPW,), jnp.int32),
        pltpu.VMEM((EPW,), jnp.int32),
        pltpu.VMEM((EPW,), jnp.float32),
        pltpu.VMEM((16, NP), jnp.float32),
        pltpu.VMEM((16, NP), jnp.float32),
        pltpu.VMEM((240, NP), jnp.float32),
        pltpu.VMEM_SHARED((240, NP), jnp.float32),
        pltpu.SemaphoreType.DMA,
    ],
)
def _sc_scatter(src_hbm, dst_hbm, ew_hbm, out_hbm, src_v, dst_v, ew_v,
                buf_w, buf_c, zer_v, shacc, sem):
    c = lax.axis_index("c")
    s = lax.axis_index("s")
    wid = c * (NWORK // 2) + s
    iota = lax.iota(jnp.int32, 16)
    z16 = jnp.zeros((16,), jnp.float32)

    @pl.when(s == 0)
    def _zinit():
        def _zrow(r, _):
            for off in range(NP // 16):
                zer_v[r, pl.ds(off * 16, 16)] = z16
            return 0
        lax.fori_loop(0, 240, _zrow, 0)
        pltpu.sync_copy(zer_v, shacc)
    plsc.subcore_barrier()

    base = wid * EPW
    pltpu.sync_copy(src_hbm.at[pl.ds(base, EPW)], src_v)
    pltpu.sync_copy(dst_hbm.at[pl.ds(base, EPW)], dst_v)
    pltpu.sync_copy(ew_hbm.at[pl.ds(base, EPW)], ew_v)

    def _chunk(j, _):
        sl = pl.ds(j * 16, 16)
        sv = src_v[sl]
        dv = dst_v[sl]
        ev = ew_v[sl]
        for l in range(16):
            lsel = jnp.full((16,), l, jnp.int32)
            sb = _vg(sv, lsel)
            eb = _vg(ev, lsel)
            for g in range(NP // 16):
                hit = (iota + (16 * g)) == sb
                buf_w[l, pl.ds(16 * g, 16)] = jnp.where(hit, eb, 0.0)
                buf_c[l, pl.ds(16 * g, 16)] = jnp.where(hit, 1.0, 0.0)
        pltpu.async_copy(buf_w, shacc.at[dv], sem, add=True).wait()
        pltpu.async_copy(buf_c, shacc.at[dv + 112], sem, add=True).wait()
        return 0

    lax.fori_loop(0, EPW // 16, _chunk, 0)
    plsc.subcore_barrier()

    @pl.when(s == 0)
    def _flush():
        pltpu.sync_copy(shacc, out_hbm.at[c])


def _fused_body(partials, xp, sidemat, lcol, rcol, wl1, wr1, wl2, wr2,
                wg1, w0, w1, w2, biasm, out_ref):
    f32 = jnp.float32
    x = xp[...]
    smat = sidemat[...]
    slab = partials[0] + partials[1]
    zrows = jnp.zeros((NP - 112, NP), f32)
    A_g = jnp.concatenate([slab[0:112], zrows], axis=0)
    A_c = jnp.concatenate([slab[112:224], zrows], axis=0)
    lvc = lcol[...]
    rvc = rcol[...]
    A_l = A_g * lvc * jnp.transpose(lvc)
    A_r = A_g * rvc * jnp.transpose(rvc)

    def gcn(A, xw):
        deg = A.sum(axis=1, keepdims=True) + 1.0
        dis = jnp.where(deg > 0, 1.0 / jnp.sqrt(jnp.abs(deg) + 1e-30), 0.0)
        return dis * (jax.lax.dot(A, dis * xw, precision=_HI) + dis * xw)

    def lrelu(t):
        return jnp.where(t >= 0, t, 0.01 * t)

    bias = biasm[...]
    lv = lvc
    rv = rvc

    h_l = lrelu(gcn(A_l, jax.lax.dot(x, wl1[...])) + bias[0:1, :])
    h_r = lrelu(gcn(A_r, jax.lax.dot(x, wr1[...])) + bias[1:2, :])
    h1 = lv * h_l + rv * h_r
    h_l2 = lrelu(gcn(A_l, jax.lax.dot(h1, wl2[...])) + bias[2:3, :])
    h_r2 = lrelu(gcn(A_r, jax.lax.dot(h1, wr2[...])) + bias[3:4, :])
    h2 = lv * h_l2 + rv * h_r2
    h2 = lrelu(gcn(A_g, jax.lax.dot(h2, wg1[...])) + bias[4:5, :])

    agg = jax.lax.dot(A_c, h2, precision=_HI)
    brel = bias[8:9, 0:1]
    raw = (jax.lax.dot(agg, smat)[:, 2:3] + brel
           + jax.lax.dot(h2, smat)[:, 3:4])
    t = jnp.exp(-2.0 * jnp.abs(raw))
    score = jnp.sign(raw) * (1.0 - t) / (1.0 + t)
    rowid = jax.lax.broadcasted_iota(jnp.int32, (NP, 1), 0)
    score = jnp.where(rowid < N_NODES, score, -3e38)

    sc_r = jnp.transpose(score)
    gt = (sc_r > score).astype(f32)
    eqlt = ((sc_r == score) & (jnp.transpose(rowid) < rowid)).astype(f32)
    rank = (gt + eqlt).sum(axis=1, keepdims=True)
    rowf = rowid.astype(f32)
    Pfull = (jnp.transpose(rank) == rowf).astype(f32)
    topv = (Pfull * jnp.transpose(score)).sum(axis=1, keepdims=True)
    selrank = (rowid < 100).astype(f32)
    pooled = jax.lax.dot(Pfull, h2, precision=_HI) * topv * selrank

    sel = (rank < 100).astype(f32)
    degf = jnp.transpose(sel) * (A_c * sel).sum(axis=0, keepdims=True)
    degv = (Pfull * degf).sum(axis=1, keepdims=True)
    disv = jnp.where(degv > 0, 1.0 / jnp.sqrt(jnp.abs(degv) + 1e-30), 0.0)
    C = A_c * sel * jnp.transpose(sel)
    Cp = jax.lax.dot_general(jax.lax.dot(Pfull, C, precision=_HI), Pfull,
                             (((1,), (1,)), ((), ())), precision=_HI)
    L = -(disv * jnp.transpose(disv)) * Cp

    Tx1 = jax.lax.dot(L, h2, precision=_HI)
    Tx2 = 2.0 * jax.lax.dot(L, Tx1, precision=_HI) - h2
    cheb_out = (jax.lax.dot(h2, w0[...])
                + jax.lax.dot(Tx1, w1[...])
                + jax.lax.dot(Tx2, w2[...]) + bias[5:6, :])

    colm = (jax.lax.broadcasted_iota(jnp.int32, (1, NP), 1) < 56).astype(f32)

    def msoftmax(z):
        zm = jnp.where(colm > 0, z, -3e38)
        e = jnp.exp(zm - zm.max(axis=1, keepdims=True)) * colm
        return e / e.sum(axis=1, keepdims=True)

    ass = msoftmax(cheb_out)
    s = msoftmax(ass) * (rowid < N_NODES).astype(f32)
    H_coarse = jax.lax.dot_general(s, h2, (((0,), (0,)), ((), ())))

    sel2 = sel * (rowid < 110).astype(f32)
    tri = (jnp.transpose(rowid) < rowid).astype(f32)
    srank = (tri * jnp.transpose(sel2)).sum(axis=1, keepdims=True)
    Q = jnp.transpose(sel2) * (jnp.transpose(srank) == rowf).astype(f32)
    rows = jax.lax.dot(Q, ass, precision=_HI)
    H1 = jax.lax.dot(rows, H_coarse)
    out_ref[...] = pooled + H1


@jax.jit
def kernel(x, edge_index, edge_attr, adj, Wl1, bl1, Wr1, br1, Wl2, bl2, Wr2,
           br2, Wg1, bg1, sag_Wrel, sag_brel, sag_Wroot, cheb_W0, cheb_W1,
           cheb_W2, cheb_b):
    f32 = jnp.float32
    pe = EP - N_EDGES
    srcp = jnp.concatenate([edge_index[0], jnp.full((pe,), N_NODES, jnp.int32)])
    dstp = jnp.concatenate([edge_index[1], jnp.full((pe,), N_NODES, jnp.int32)])
    ewp = jnp.concatenate([edge_attr, jnp.zeros((pe,), f32)])

    partials = _sc_scatter(srcp, dstp, ewp)

    def pad2(a):
        return jnp.zeros((NP, NP), f32).at[:a.shape[0], :a.shape[1]].set(a)

    sideL = np.zeros((NP,), np.float32); sideL[_LEFT] = 1.0
    sideR = np.zeros((NP,), np.float32); sideR[_RIGHT] = 1.0
    sidemat = np.zeros((NP, NP), np.float32)
    sidemat[:, 0] = sideL; sidemat[:, 1] = sideR
    sidemat = jnp.asarray(sidemat)
    sidemat = sidemat.at[:20, 2].set(sag_Wrel[:, 0])
    sidemat = sidemat.at[:20, 3].set(sag_Wroot[:, 0])

    biasm = jnp.zeros((16, NP), f32)
    biasm = biasm.at[0, :64].set(bl1).at[1, :64].set(br1)
    biasm = biasm.at[2, :20].set(bl2).at[3, :20].set(br2)
    biasm = biasm.at[4, :20].set(bg1).at[5, :56].set(cheb_b)
    biasm = biasm.at[8, 0].set(sag_brel[0])

    out = pl.pallas_call(
        _fused_body,
        out_shape=jax.ShapeDtypeStruct((NP, NP), f32),
    )(
        partials, pad2(x), sidemat, jnp.asarray(sideL)[:, None],
        jnp.asarray(sideR)[:, None], pad2(Wl1), pad2(Wr1), pad2(Wl2),
        pad2(Wr2), pad2(Wg1), pad2(cheb_W0), pad2(cheb_W1), pad2(cheb_W2),
        biasm,
    )
    return out[:100, :20].reshape(1, -1)
